# baseline (speedup 1.0000x reference)
import dataclasses
import functools

import jax
import jax.numpy as jnp
from jax import lax
from jax.experimental import pallas as pl
from jax.experimental.pallas import tpu as pltpu
from jax.experimental.pallas import tpu_sc as plsc

N = 10000
N_PAD = 10240
D = 128
NC = 2
NS = 16
GROUP = 128
NB0 = 160
NB1 = 4
E_C0 = NB0 * GROUP * NS
E_PAD = E_C0 + NB1 * GROUP * NS
ROWS_SLICE = N_PAD // NS

_sc_params = pltpu.CompilerParams()
if "needs_layout_passes" in pltpu.CompilerParams.__dataclass_fields__:
    _sc_params = dataclasses.replace(_sc_params, needs_layout_passes=False)


@functools.partial(
    pl.kernel,
    compiler_params=_sc_params,
    out_type=jax.ShapeDtypeStruct((NC, N_PAD, D), jnp.float32),
    mesh=plsc.VectorSubcoreMesh(core_axis_name="c", subcore_axis_name="s"),
    scratch_types=(
        [pltpu.VMEM_SHARED((N_PAD, D), jnp.float32)]
        + [pltpu.VMEM((GROUP, D), jnp.float32)] * 2
        + [pltpu.VMEM((1, GROUP), jnp.int32)] * 4
        + [pltpu.VMEM((1, GROUP), jnp.int32)] * 4
        + [pltpu.VMEM((GROUP,), jnp.float32)] * 4
        + [pltpu.SemaphoreType.DMA] * 6
    ),
)
def _sc_spmm(cur_hbm, src_hbm, dst_hbm, val_hbm, out_hbm,
             acc, rows0, rows1, si0, si1, si2, si3, di0, di1, di2, di3,
             va0, va1, va2, va3, sg0, sg1, sj0, sj1, sj2, sj3):
    c = lax.axis_index("c")
    s = lax.axis_index("s")
    base = s * ROWS_SLICE
    nb = jnp.where(c == 0, NB0, NB1)
    e0 = jnp.where(c == 0, s * (NB0 * GROUP), E_C0 + s * (NB1 * GROUP))
    g0 = e0 // GROUP

    rows = (rows0, rows1)
    sidx = (si0, si1, si2, si3)
    didx = (di0, di1, di2, di3)
    vals = (va0, va1, va2, va3)
    sg = (sg0, sg1)
    sj = (sj0, sj1, sj2, sj3)

    def idx_copies(b, k):
        return (
            pltpu.make_async_copy(src_hbm.at[pl.ds(g0 + b, 1)], sidx[k], sj[k]),
            pltpu.make_async_copy(dst_hbm.at[pl.ds(g0 + b, 1)], didx[k], sj[k]),
            pltpu.make_async_copy(val_hbm.at[pl.ds(e0 + b * GROUP, GROUP)],
                                  vals[k], sj[k]),
        )

    def start_idx(b, k):
        for cp in idx_copies(b, k):
            cp.start()

    def wait_idx(b, k):
        for cp in idx_copies(b, k):
            cp.wait()

    def gather_copy(p, k):
        return pltpu.make_async_copy(cur_hbm.at[sidx[k].at[0]], rows[p], sg[p])

    zero = jnp.zeros((16,), jnp.float32)

    @pl.loop(0, GROUP)
    def _(i):
        for j in range(D // 16):
            rows0[i, pl.ds(j * 16, 16)] = zero

    zcopies = [
        pltpu.make_async_copy(rows0, acc.at[pl.ds(base + t * GROUP, GROUP)], sg0)
        for t in range(ROWS_SLICE // GROUP)
    ]
    for cp in zcopies:
        cp.start()
    for k in range(4):
        start_idx(k, k)
    for cp in zcopies:
        cp.wait()
    plsc.subcore_barrier()

    def scale(p, k):
        rb = rows[p]
        vb = vals[k]

        @pl.loop(0, GROUP)
        def _(i):
            iv = jnp.full((16,), i, jnp.int32)
            v = plsc.load_gather(vb, [iv])
            for j in range(D // 16):
                sl = pl.ds(j * 16, 16)
                rb[i, sl] = rb[i, sl] * v

    def step(b, k, p, next_gather, next_idx):
        gather_copy(p, k).wait()
        scale(p, k)
        pltpu.sync_copy(rows[p], acc.at[didx[k].at[0]], add=True)
        if next_idx:
            start_idx(b + 4, k)
        if next_gather:
            kn = (k + 2) % 4
            wait_idx(b + 2, kn)
            gather_copy(p, kn).start()

    wait_idx(0, 0)
    gather_copy(0, 0).start()
    wait_idx(1, 1)
    gather_copy(1, 1).start()

    @pl.loop(0, (nb - 4) // 4)
    def _(t):
        b = t * 4
        step(b + 0, 0, 0, True, True)
        step(b + 1, 1, 1, True, True)
        step(b + 2, 2, 0, True, True)
        step(b + 3, 3, 1, True, True)

    step(nb - 4, 0, 0, True, False)
    step(nb - 3, 1, 1, True, False)
    step(nb - 2, 2, 0, False, False)
    step(nb - 1, 3, 1, False, False)

    plsc.subcore_barrier()
    pltpu.sync_copy(acc.at[pl.ds(base, ROWS_SLICE)],
                    out_hbm.at[c, pl.ds(base, ROWS_SLICE)])


def _final_body(h_ref, c1_ref, c2_ref, c3_ref, out_ref):
    out_ref[...] = (h_ref[...] + c1_ref[0] + c2_ref[0] + c3_ref[0]) * 0.25


_tc_final = pl.pallas_call(
    _final_body,
    out_shape=jax.ShapeDtypeStruct((N_PAD, D), jnp.float32),
)


def kernel(h, edge_index, adj_values):
    src = edge_index[1].astype(jnp.int32)
    dst = edge_index[0].astype(jnp.int32)
    e = src.shape[0]
    pad = E_PAD - e
    src2d = jnp.concatenate([src, jnp.zeros((pad,), jnp.int32)]).reshape(-1, GROUP)
    dst2d = jnp.concatenate([dst, jnp.zeros((pad,), jnp.int32)]).reshape(-1, GROUP)
    val1d = jnp.concatenate(
        [adj_values.astype(jnp.float32), jnp.zeros((pad,), jnp.float32)])

    hp = jnp.pad(h, ((0, N_PAD - h.shape[0]), (0, 0)))
    c1 = _sc_spmm(hp, src2d, dst2d, val1d)
    c2 = _sc_spmm(c1[0], src2d, dst2d, val1d)
    c3 = _sc_spmm(c2[0], src2d, dst2d, val1d)
    out = _tc_final(hp, c1, c2, c3)
    return out[:h.shape[0]]


# speedup vs baseline: 1.6287x; 1.6287x over previous
import dataclasses
import functools

import jax
import jax.numpy as jnp
from jax import lax
from jax.experimental import pallas as pl
from jax.experimental.pallas import tpu as pltpu
from jax.experimental.pallas import tpu_sc as plsc

N = 10000
N_PAD = 10240
D = 128
NC = 2
NS = 16
GROUP = 128
NB0 = 120
NB1 = 40
E_C0 = NB0 * GROUP * NS
E_PAD = E_C0 + NB1 * GROUP * NS
ROWS_SLICE = N_PAD // NS

_sc_params = pltpu.CompilerParams()
if "needs_layout_passes" in pltpu.CompilerParams.__dataclass_fields__:
    _sc_params = dataclasses.replace(_sc_params, needs_layout_passes=False)


@functools.partial(
    pl.kernel,
    compiler_params=_sc_params,
    out_type=jax.ShapeDtypeStruct((NC, N_PAD, D), jnp.float32),
    mesh=plsc.VectorSubcoreMesh(core_axis_name="c", subcore_axis_name="s"),
    scratch_types=(
        [pltpu.VMEM_SHARED((N_PAD, D), jnp.float32)]
        + [pltpu.VMEM((GROUP, D), jnp.float32)] * 2
        + [pltpu.VMEM((1, GROUP), jnp.int32)] * 4
        + [pltpu.VMEM((1, GROUP), jnp.int32)] * 4
        + [pltpu.VMEM((GROUP,), jnp.float32)] * 4
        + [pltpu.SemaphoreType.DMA] * 6
    ),
)
def _sc_spmm(cur_hbm, src_hbm, dst_hbm, val_hbm, out_hbm,
             acc, rows0, rows1, si0, si1, si2, si3, di0, di1, di2, di3,
             va0, va1, va2, va3, sg0, sg1, sj0, sj1, sj2, sj3):
    c = lax.axis_index("c")
    s = lax.axis_index("s")
    base = s * ROWS_SLICE
    nb = jnp.where(c == 0, NB0, NB1)
    e0 = jnp.where(c == 0, s * (NB0 * GROUP), E_C0 + s * (NB1 * GROUP))
    g0 = e0 // GROUP

    rows = (rows0, rows1)
    sidx = (si0, si1, si2, si3)
    didx = (di0, di1, di2, di3)
    vals = (va0, va1, va2, va3)
    sg = (sg0, sg1)
    sj = (sj0, sj1, sj2, sj3)

    def idx_copies(b, k):
        return (
            pltpu.make_async_copy(src_hbm.at[pl.ds(g0 + b, 1)], sidx[k], sj[k]),
            pltpu.make_async_copy(dst_hbm.at[pl.ds(g0 + b, 1)], didx[k], sj[k]),
            pltpu.make_async_copy(val_hbm.at[pl.ds(e0 + b * GROUP, GROUP)],
                                  vals[k], sj[k]),
        )

    def start_idx(b, k):
        for cp in idx_copies(b, k):
            cp.start()

    def wait_idx(b, k):
        for cp in idx_copies(b, k):
            cp.wait()

    def gather_copy(p, k):
        return pltpu.make_async_copy(cur_hbm.at[sidx[k].at[0]], rows[p], sg[p])

    zero = jnp.zeros((16,), jnp.float32)

    @pl.loop(0, GROUP)
    def _(i):
        for j in range(D // 16):
            rows0[i, pl.ds(j * 16, 16)] = zero

    zcopies = [
        pltpu.make_async_copy(rows0, acc.at[pl.ds(base + t * GROUP, GROUP)], sg0)
        for t in range(ROWS_SLICE // GROUP)
    ]
    for cp in zcopies:
        cp.start()
    for k in range(4):
        start_idx(k, k)
    for cp in zcopies:
        cp.wait()
    plsc.subcore_barrier()

    def scale(p, k):
        rb = rows[p]
        vb = vals[k]

        @plsc.parallel_loop(0, GROUP, step=1, unroll=4)
        def _(i):
            iv = jnp.full((16,), i, jnp.int32)
            v = plsc.load_gather(vb, [iv])
            for j in range(D // 16):
                sl = pl.ds(j * 16, 16)
                rb[i, sl] = rb[i, sl] * v

    def step(b, k, p, next_gather, next_idx):
        gather_copy(p, k).wait()
        scale(p, k)
        pltpu.sync_copy(rows[p], acc.at[didx[k].at[0]], add=True)
        if next_idx:
            start_idx(b + 4, k)
        if next_gather:
            kn = (k + 2) % 4
            wait_idx(b + 2, kn)
            gather_copy(p, kn).start()

    wait_idx(0, 0)
    gather_copy(0, 0).start()
    wait_idx(1, 1)
    gather_copy(1, 1).start()

    @pl.loop(0, (nb - 4) // 4)
    def _(t):
        b = t * 4
        step(b + 0, 0, 0, True, True)
        step(b + 1, 1, 1, True, True)
        step(b + 2, 2, 0, True, True)
        step(b + 3, 3, 1, True, True)

    step(nb - 4, 0, 0, True, False)
    step(nb - 3, 1, 1, True, False)
    step(nb - 2, 2, 0, False, False)
    step(nb - 1, 3, 1, False, False)

    plsc.subcore_barrier()
    pltpu.sync_copy(acc.at[pl.ds(base, ROWS_SLICE)],
                    out_hbm.at[c, pl.ds(base, ROWS_SLICE)])


def _combine_body(parts_ref, tot_ref, cur_out, tot_out):
    p = parts_ref[0] + parts_ref[1]
    cur_out[...] = p
    tot_out[...] = tot_ref[...] + p


_tc_combine = pl.pallas_call(
    _combine_body,
    out_shape=[jax.ShapeDtypeStruct((N_PAD, D), jnp.float32)] * 2,
)


def _final_body(parts_ref, tot_ref, out_ref):
    p = parts_ref[0] + parts_ref[1]
    out_ref[...] = (tot_ref[...] + p) * 0.25


_tc_final = pl.pallas_call(
    _final_body,
    out_shape=jax.ShapeDtypeStruct((N_PAD, D), jnp.float32),
)


def kernel(h, edge_index, adj_values):
    src = edge_index[1].astype(jnp.int32)
    dst = edge_index[0].astype(jnp.int32)
    e = src.shape[0]
    pad = E_PAD - e
    src2d = jnp.concatenate([src, jnp.zeros((pad,), jnp.int32)]).reshape(-1, GROUP)
    dst2d = jnp.concatenate([dst, jnp.zeros((pad,), jnp.int32)]).reshape(-1, GROUP)
    val1d = jnp.concatenate(
        [adj_values.astype(jnp.float32), jnp.zeros((pad,), jnp.float32)])

    hp = jnp.pad(h, ((0, N_PAD - h.shape[0]), (0, 0)))
    cur = hp
    tot = hp
    for layer in range(3):
        parts = _sc_spmm(cur, src2d, dst2d, val1d)
        if layer < 2:
            cur, tot = _tc_combine(parts, tot)
        else:
            out = _tc_final(parts, tot)
    return out[:h.shape[0]]


# speedup vs baseline: 1.6390x; 1.0063x over previous
import dataclasses
import functools

import jax
import jax.numpy as jnp
from jax import lax
from jax.experimental import pallas as pl
from jax.experimental.pallas import tpu as pltpu
from jax.experimental.pallas import tpu_sc as plsc

N = 10000
N_PAD = 10240
D = 128
NC = 2
NS = 16
GROUP = 128
NB0 = 120
NB1 = 40
E_C0 = NB0 * GROUP * NS
E_PAD = E_C0 + NB1 * GROUP * NS
ROWS_SLICE = N_PAD // NS

_sc_params = pltpu.CompilerParams()
if "needs_layout_passes" in pltpu.CompilerParams.__dataclass_fields__:
    _sc_params = dataclasses.replace(_sc_params, needs_layout_passes=False)


@functools.partial(
    pl.kernel,
    compiler_params=_sc_params,
    out_type=jax.ShapeDtypeStruct((NC, N_PAD, D), jnp.float32),
    mesh=plsc.VectorSubcoreMesh(core_axis_name="c", subcore_axis_name="s"),
    scratch_types=(
        [pltpu.VMEM_SHARED((N_PAD, D), jnp.float32)]
        + [pltpu.VMEM((GROUP, D), jnp.float32)] * 2
        + [pltpu.VMEM((1, GROUP), jnp.int32)] * 4
        + [pltpu.VMEM((1, GROUP), jnp.int32)] * 4
        + [pltpu.VMEM((GROUP,), jnp.float32)] * 4
        + [pltpu.SemaphoreType.DMA] * 6
    ),
)
def _sc_spmm(cur_hbm, src_hbm, dst_hbm, val_hbm, out_hbm,
             acc, rows0, rows1, si0, si1, si2, si3, di0, di1, di2, di3,
             va0, va1, va2, va3, sg0, sg1, sj0, sj1, sj2, sj3):
    c = lax.axis_index("c")
    s = lax.axis_index("s")
    base = s * ROWS_SLICE
    nb = jnp.where(c == 0, NB0, NB1)
    e0 = jnp.where(c == 0, s * (NB0 * GROUP), E_C0 + s * (NB1 * GROUP))
    g0 = e0 // GROUP

    rows = (rows0, rows1)
    sidx = (si0, si1, si2, si3)
    didx = (di0, di1, di2, di3)
    vals = (va0, va1, va2, va3)
    sg = (sg0, sg1)
    sj = (sj0, sj1, sj2, sj3)

    def idx_copies(b, k):
        return (
            pltpu.make_async_copy(src_hbm.at[pl.ds(g0 + b, 1)], sidx[k], sj[k]),
            pltpu.make_async_copy(dst_hbm.at[pl.ds(g0 + b, 1)], didx[k], sj[k]),
            pltpu.make_async_copy(val_hbm.at[pl.ds(e0 + b * GROUP, GROUP)],
                                  vals[k], sj[k]),
        )

    def start_idx(b, k):
        for cp in idx_copies(b, k):
            cp.start()

    def wait_idx(b, k):
        for cp in idx_copies(b, k):
            cp.wait()

    def gather_copy(p, k):
        return pltpu.make_async_copy(cur_hbm.at[sidx[k].at[0]], rows[p], sg[p])

    zero = jnp.zeros((16,), jnp.float32)

    @pl.loop(0, GROUP)
    def _(i):
        for j in range(D // 16):
            rows0[i, pl.ds(j * 16, 16)] = zero

    zcopies = [
        pltpu.make_async_copy(rows0, acc.at[pl.ds(base + t * GROUP, GROUP)], sg0)
        for t in range(ROWS_SLICE // GROUP)
    ]
    for cp in zcopies:
        cp.start()
    for k in range(4):
        start_idx(k, k)
    for cp in zcopies:
        cp.wait()
    plsc.subcore_barrier()

    def scale(p, k):
        rb = rows[p]
        vb = vals[k]

        @pl.loop(0, GROUP)
        def _(i):
            iv = jnp.full((16,), i, jnp.int32)
            v = plsc.load_gather(vb, [iv])
            for j in range(D // 16):
                sl = pl.ds(j * 16, 16)
                rb[i, sl] = rb[i, sl] * v

    def step(b, k, p, next_gather, next_idx):
        gather_copy(p, k).wait()
        scale(p, k)
        pltpu.sync_copy(rows[p], acc.at[didx[k].at[0]], add=True)
        if next_idx:
            start_idx(b + 4, k)
        if next_gather:
            kn = (k + 2) % 4
            wait_idx(b + 2, kn)
            gather_copy(p, kn).start()

    wait_idx(0, 0)
    gather_copy(0, 0).start()
    wait_idx(1, 1)
    gather_copy(1, 1).start()

    @pl.loop(0, (nb - 4) // 4)
    def _(t):
        b = t * 4
        step(b + 0, 0, 0, True, True)
        step(b + 1, 1, 1, True, True)
        step(b + 2, 2, 0, True, True)
        step(b + 3, 3, 1, True, True)

    step(nb - 4, 0, 0, True, False)
    step(nb - 3, 1, 1, True, False)
    step(nb - 2, 2, 0, False, False)
    step(nb - 1, 3, 1, False, False)

    plsc.subcore_barrier()
    pltpu.sync_copy(acc.at[pl.ds(base, ROWS_SLICE)],
                    out_hbm.at[c, pl.ds(base, ROWS_SLICE)])


def _combine_body(parts_ref, tot_ref, cur_out, tot_out):
    p = parts_ref[0] + parts_ref[1]
    cur_out[...] = p
    tot_out[...] = tot_ref[...] + p


_tc_combine = pl.pallas_call(
    _combine_body,
    out_shape=[jax.ShapeDtypeStruct((N_PAD, D), jnp.float32)] * 2,
)


def _final_body(parts_ref, tot_ref, out_ref):
    p = parts_ref[0] + parts_ref[1]
    out_ref[...] = (tot_ref[...] + p) * 0.25


_tc_final = pl.pallas_call(
    _final_body,
    out_shape=jax.ShapeDtypeStruct((N_PAD, D), jnp.float32),
)


def kernel(h, edge_index, adj_values):
    src = edge_index[1].astype(jnp.int32)
    dst = edge_index[0].astype(jnp.int32)
    e = src.shape[0]
    pad = E_PAD - e
    src2d = jnp.concatenate([src, jnp.zeros((pad,), jnp.int32)]).reshape(-1, GROUP)
    dst2d = jnp.concatenate([dst, jnp.zeros((pad,), jnp.int32)]).reshape(-1, GROUP)
    val1d = jnp.concatenate(
        [adj_values.astype(jnp.float32), jnp.zeros((pad,), jnp.float32)])

    hp = jnp.pad(h, ((0, N_PAD - h.shape[0]), (0, 0)))
    cur = hp
    tot = hp
    for layer in range(3):
        parts = _sc_spmm(cur, src2d, dst2d, val1d)
        if layer < 2:
            cur, tot = _tc_combine(parts, tot)
        else:
            out = _tc_final(parts, tot)
    return out[:h.shape[0]]
